# baseline (speedup 1.0000x reference)
import functools

import jax
import jax.numpy as jnp
from jax import lax
from jax.experimental import pallas as pl
from jax.experimental.pallas import tpu as pltpu
from jax.experimental.pallas import tpu_sc as plsc

NSUB = 32
K = 96


def _sc_aggregate(h, src, dst_full, dst_tail, zeros_tile):
    n, d = h.shape
    nfull = dst_full.shape[1]
    ktail = dst_tail.shape[1]
    epw = nfull * K + ktail
    npad = zeros_tile.shape[0] * 16
    rows_per_tile = npad // 16
    nlane = 2
    nstep = nfull // nlane

    mesh = plsc.VectorSubcoreMesh(core_axis_name="c", subcore_axis_name="s")

    @functools.partial(
        pl.kernel,
        out_type=jax.ShapeDtypeStruct((2, npad, d), jnp.float32),
        mesh=mesh,
        scratch_types=[
            pltpu.VMEM_SHARED((npad, d), jnp.float32),
            pltpu.VMEM((epw,), jnp.int32),
            pltpu.VMEM((nfull, K), jnp.int32),
            pltpu.VMEM((ktail,), jnp.int32),
            [pltpu.VMEM((K, d), jnp.float32)] * 2,
            [pltpu.SemaphoreType.DMA] * 2,
            [pltpu.SemaphoreType.DMA] * 2,
        ],
    )
    def agg_kernel(h_hbm, src_hbm, dst_hbm, dstt_hbm, z_hbm, out_hbm,
                   acc, src_v, dst_v, dstt_v, rows, gsems, ssems):
        c = lax.axis_index("c")
        s = lax.axis_index("s")
        wid = s * 2 + c
        r0 = pl.multiple_of(s * rows_per_tile, 8)

        pltpu.sync_copy(z_hbm, acc.at[pl.ds(r0, rows_per_tile)])
        pltpu.sync_copy(src_hbm.at[pl.ds(pl.multiple_of(wid * epw, 8), epw)],
                        src_v)
        pltpu.sync_copy(dst_hbm.at[wid], dst_v)
        pltpu.sync_copy(dstt_hbm.at[wid], dstt_v)
        plsc.subcore_barrier()

        def gath(j, buf, sem):
            off = pl.multiple_of(j * K, 8)
            pltpu.async_copy(h_hbm.at[src_v.at[pl.ds(off, K)]], buf, sem)

        def wait_g(buf, sem):
            pltpu.make_async_copy(h_hbm.at[src_v.at[pl.ds(0, K)]],
                                  buf, sem).wait()

        def wait_s(buf, sem):
            pltpu.make_async_copy(buf, acc.at[dst_v.at[0]], sem).wait()

        for l in range(nlane):
            gath(l, rows[l], gsems[l])

        def step(j3, carry):
            j0 = j3 * nlane
            for l in range(nlane):
                wait_g(rows[l], gsems[l])
                pltpu.async_copy(rows[l], acc.at[dst_v.at[j0 + l]],
                                 ssems[l], add=True)
            for l in range(nlane):
                @pl.when(j0 + l + nlane < nfull)
                def _(l=l):
                    wait_s(rows[l], ssems[l])
                    gath(j0 + l + nlane, rows[l], gsems[l])
            return carry

        lax.fori_loop(0, nstep, step, 0)

        for l in range(nlane):
            wait_s(rows[l], ssems[l])

        toff = pl.multiple_of(nfull * K, 8)
        pltpu.async_copy(h_hbm.at[src_v.at[pl.ds(toff, ktail)]],
                         rows[0].at[pl.ds(0, ktail)], gsems[0]).wait()
        pltpu.sync_copy(rows[0].at[pl.ds(0, ktail)], acc.at[dstt_v], add=True)

        plsc.subcore_barrier()
        pltpu.sync_copy(acc.at[pl.ds(r0, rows_per_tile)],
                        out_hbm.at[c].at[pl.ds(r0, rows_per_tile)])

    return agg_kernel(h, src, dst_full, dst_tail, zeros_tile)


def _tc_layer(xin, agg, w1, b1, w2, b2, gamma, beta, starts):
    n, d = xin.shape
    nh = w1.shape[1]
    g_num = starts.shape[0] - 1
    neg_inf = float('-inf')

    def body(x_ref, agg_ref, w1_ref, b1_ref, w2_ref, b2_ref, ga_ref, be_ref,
             st_ref, h_ref, pool_ref):
        hin = x_ref[...] + agg_ref[0, :n, :] + agg_ref[1, :n, :]
        t = jnp.maximum(
            jnp.dot(hin, w1_ref[...], preferred_element_type=jnp.float32)
            + b1_ref[...], 0.0)
        u = (jnp.dot(t, w2_ref[...], preferred_element_type=jnp.float32)
             + b2_ref[...])
        mu = jnp.mean(u, axis=0, keepdims=True)
        var = jnp.mean((u - mu) * (u - mu), axis=0, keepdims=True)
        h = jnp.maximum(
            ga_ref[...] * (u - mu) * lax.rsqrt(var + 1e-5) + be_ref[...], 0.0)
        h_ref[...] = h

        def seg_group(gg, carry):
            rows_out = []
            for k8 in range(8):
                gi = gg * 8 + k8
                s0 = st_ref[gi]
                e0 = st_ref[gi + 1]
                c0 = s0 // 8
                c1 = (e0 + 7) // 8

                def chunk(ci, acc):
                    mx, sm = acc
                    rows = h_ref[pl.ds(pl.multiple_of(ci * 8, 8), 8), :]
                    ridx = ci * 8 + lax.broadcasted_iota(jnp.int32, (8, 1), 0)
                    m = (ridx >= s0) & (ridx < e0)
                    mx = jnp.maximum(mx, jnp.where(m, rows, neg_inf))
                    sm = sm + jnp.where(m, rows, 0.0)
                    return (mx, sm)

                mx, sm = lax.fori_loop(
                    c0, c1, chunk,
                    (jnp.full((8, nh), neg_inf, jnp.float32),
                     jnp.zeros((8, nh), jnp.float32)))
                mxr = jnp.max(mx, axis=0, keepdims=True)
                smr = jnp.sum(sm, axis=0, keepdims=True)
                cnt = jnp.maximum((e0 - s0).astype(jnp.float32), 1.0)
                rows_out.append(jnp.concatenate([mxr, smr / cnt], axis=1))
            blk = jnp.concatenate(rows_out, axis=0)
            pool_ref[pl.ds(pl.multiple_of(gg * 8, 8), 8), :] = blk
            return carry

        lax.fori_loop(0, g_num // 8, seg_group, 0)

    return pl.pallas_call(
        body,
        out_shape=(jax.ShapeDtypeStruct((n, nh), jnp.float32),
                   jax.ShapeDtypeStruct((g_num, 2 * nh), jnp.float32)),
        in_specs=[
            pl.BlockSpec(memory_space=pltpu.VMEM),
            pl.BlockSpec(memory_space=pltpu.VMEM),
            pl.BlockSpec(memory_space=pltpu.VMEM),
            pl.BlockSpec(memory_space=pltpu.VMEM),
            pl.BlockSpec(memory_space=pltpu.VMEM),
            pl.BlockSpec(memory_space=pltpu.VMEM),
            pl.BlockSpec(memory_space=pltpu.VMEM),
            pl.BlockSpec(memory_space=pltpu.VMEM),
            pl.BlockSpec(memory_space=pltpu.SMEM),
        ],
        out_specs=(pl.BlockSpec(memory_space=pltpu.VMEM),
                   pl.BlockSpec(memory_space=pltpu.VMEM)),
    )(xin, agg, w1, b1, w2, b2, gamma, beta, starts)


def _head(pools, l1, l2, l3):
    p = pools[0] + pools[1]
    q = pools[2] + pools[3]
    ng = l3['w'].shape[1]
    ng_pad = ((ng + 127) // 128) * 128
    w3 = jnp.pad(l3['w'], ((0, 0), (0, ng_pad - ng)))
    b3 = jnp.pad(l3['b'], (0, ng_pad - ng)).reshape(1, -1)

    def body(p_ref, q_ref, w1_ref, b1_ref, w2_ref, b2_ref, w3_ref, b3_ref,
             y_ref):
        z = p_ref[...] + q_ref[...]
        z = jnp.maximum(
            jnp.dot(z, w1_ref[...], preferred_element_type=jnp.float32)
            + b1_ref[...], 0.0)
        z = jnp.maximum(
            jnp.dot(z, w2_ref[...], preferred_element_type=jnp.float32)
            + b2_ref[...], 0.0)
        y_ref[...] = (jnp.dot(z, w3_ref[...],
                              preferred_element_type=jnp.float32)
                      + b3_ref[...])

    y = pl.pallas_call(
        body,
        out_shape=jax.ShapeDtypeStruct((pools[0].shape[0], ng_pad),
                                       jnp.float32),
    )(p, q, l1['w'], l1['b'].reshape(1, -1), l2['w'], l2['b'].reshape(1, -1),
      w3, b3)
    return y[:, :ng]


def kernel(x, params, edge_index, batch):
    n, d = x.shape
    e = edge_index.shape[1]
    g_num = 128
    epw = e // NSUB
    nfull = (epw // K) // 2 * 2
    ktail = epw - nfull * K
    npad = ((n + 127) // 128) * 128

    src = edge_index[0]
    dst_w = edge_index[1].reshape(NSUB, epw)
    dst_full = dst_w[:, :nfull * K].reshape(NSUB, nfull, K)
    dst_tail = dst_w[:, nfull * K:]
    zeros_tile = jnp.zeros((npad // 16, d), jnp.float32)

    counts = jnp.sum(
        batch[None, :] == jnp.arange(g_num, dtype=jnp.int32)[:, None],
        axis=1, dtype=jnp.int32)
    starts = jnp.concatenate(
        [jnp.zeros((1,), jnp.int32), jnp.cumsum(counts, dtype=jnp.int32)])

    h = x
    pools = []
    for l in range(4):
        cp = params['conv%d' % l]
        bn = params['bn%d' % l]
        agg = _sc_aggregate(h, src, dst_full, dst_tail, zeros_tile)
        h, pool = _tc_layer(h, agg, cp['w1'], cp['b1'].reshape(1, -1),
                            cp['w2'], cp['b2'].reshape(1, -1),
                            bn['gamma'].reshape(1, -1),
                            bn['beta'].reshape(1, -1), starts)
        pools.append(pool)
    return _head(pools, params['lin1'], params['lin2'], params['lin3'])


# speedup vs baseline: 1.2519x; 1.2519x over previous
import functools

import jax
import jax.numpy as jnp
from jax import lax
from jax.experimental import pallas as pl
from jax.experimental.pallas import tpu as pltpu
from jax.experimental.pallas import tpu_sc as plsc

NSUB = 32
K = 96


def _sc_aggregate(h, src, dst_full, dst_tail, zeros_tile):
    n, d = h.shape
    nfull = dst_full.shape[1]
    ktail = dst_tail.shape[1]
    epw = nfull * K + ktail
    npad = zeros_tile.shape[0] * 16
    rows_per_tile = npad // 16
    nlane = 2
    nstep = nfull // nlane

    mesh = plsc.VectorSubcoreMesh(core_axis_name="c", subcore_axis_name="s")

    @functools.partial(
        pl.kernel,
        out_type=jax.ShapeDtypeStruct((2, npad, d), jnp.float32),
        mesh=mesh,
        scratch_types=[
            pltpu.VMEM_SHARED((npad, d), jnp.float32),
            pltpu.VMEM((epw,), jnp.int32),
            pltpu.VMEM((nfull, K), jnp.int32),
            pltpu.VMEM((ktail,), jnp.int32),
            [pltpu.VMEM((K, d), jnp.float32)] * 2,
            [pltpu.SemaphoreType.DMA] * 2,
            [pltpu.SemaphoreType.DMA] * 2,
        ],
    )
    def agg_kernel(h_hbm, src_hbm, dst_hbm, dstt_hbm, z_hbm, out_hbm,
                   acc, src_v, dst_v, dstt_v, rows, gsems, ssems):
        c = lax.axis_index("c")
        s = lax.axis_index("s")
        wid = s * 2 + c
        r0 = pl.multiple_of(s * rows_per_tile, 8)

        pltpu.sync_copy(z_hbm, acc.at[pl.ds(r0, rows_per_tile)])
        pltpu.sync_copy(src_hbm.at[pl.ds(pl.multiple_of(wid * epw, 8), epw)],
                        src_v)
        pltpu.sync_copy(dst_hbm.at[wid], dst_v)
        pltpu.sync_copy(dstt_hbm.at[wid], dstt_v)
        plsc.subcore_barrier()

        def gath(j, buf, sem):
            off = pl.multiple_of(j * K, 8)
            pltpu.async_copy(h_hbm.at[src_v.at[pl.ds(off, K)]], buf, sem)

        def wait_g(buf, sem):
            pltpu.make_async_copy(h_hbm.at[src_v.at[pl.ds(0, K)]],
                                  buf, sem).wait()

        def wait_s(buf, sem):
            pltpu.make_async_copy(buf, acc.at[dst_v.at[0]], sem).wait()

        for l in range(nlane):
            gath(l, rows[l], gsems[l])

        def step(j3, carry):
            j0 = j3 * nlane
            for l in range(nlane):
                wait_g(rows[l], gsems[l])
                pltpu.sync_copy(rows[l], acc.at[dst_v.at[j0 + l]], add=True)

                @pl.when(j0 + l + nlane < nfull)
                def _(l=l):
                    gath(j0 + l + nlane, rows[l], gsems[l])
            return carry

        lax.fori_loop(0, nstep, step, 0)

        toff = pl.multiple_of(nfull * K, 8)
        pltpu.async_copy(h_hbm.at[src_v.at[pl.ds(toff, ktail)]],
                         rows[0].at[pl.ds(0, ktail)], gsems[0]).wait()
        pltpu.sync_copy(rows[0].at[pl.ds(0, ktail)], acc.at[dstt_v], add=True)

        plsc.subcore_barrier()
        pltpu.sync_copy(acc.at[pl.ds(r0, rows_per_tile)],
                        out_hbm.at[c].at[pl.ds(r0, rows_per_tile)])

    return agg_kernel(h, src, dst_full, dst_tail, zeros_tile)


def _tc_layer(xin, agg, w1, b1, w2, b2, gamma, beta, starts):
    n, d = xin.shape
    nh = w1.shape[1]
    g_num = starts.shape[0] - 1
    neg_inf = float('-inf')

    def body(x_ref, agg_ref, w1_ref, b1_ref, w2_ref, b2_ref, ga_ref, be_ref,
             st_ref, h_ref, pool_ref):
        hin = x_ref[...] + agg_ref[0, :n, :] + agg_ref[1, :n, :]
        t = jnp.maximum(
            jnp.dot(hin, w1_ref[...], preferred_element_type=jnp.float32)
            + b1_ref[...], 0.0)
        u = (jnp.dot(t, w2_ref[...], preferred_element_type=jnp.float32)
             + b2_ref[...])
        mu = jnp.mean(u, axis=0, keepdims=True)
        var = jnp.mean((u - mu) * (u - mu), axis=0, keepdims=True)
        h = jnp.maximum(
            ga_ref[...] * (u - mu) * lax.rsqrt(var + 1e-5) + be_ref[...], 0.0)
        h_ref[...] = h

        def seg_group(gg, carry):
            rows_out = []
            for k8 in range(8):
                gi = gg * 8 + k8
                s0 = st_ref[gi]
                e0 = st_ref[gi + 1]
                c0 = s0 // 8
                c1 = (e0 + 7) // 8

                def chunk(ci, acc):
                    mx, sm = acc
                    rows = h_ref[pl.ds(pl.multiple_of(ci * 8, 8), 8), :]
                    ridx = ci * 8 + lax.broadcasted_iota(jnp.int32, (8, 1), 0)
                    m = (ridx >= s0) & (ridx < e0)
                    mx = jnp.maximum(mx, jnp.where(m, rows, neg_inf))
                    sm = sm + jnp.where(m, rows, 0.0)
                    return (mx, sm)

                mx, sm = lax.fori_loop(
                    c0, c1, chunk,
                    (jnp.full((8, nh), neg_inf, jnp.float32),
                     jnp.zeros((8, nh), jnp.float32)))
                mxr = jnp.max(mx, axis=0, keepdims=True)
                smr = jnp.sum(sm, axis=0, keepdims=True)
                cnt = jnp.maximum((e0 - s0).astype(jnp.float32), 1.0)
                rows_out.append(jnp.concatenate([mxr, smr / cnt], axis=1))
            blk = jnp.concatenate(rows_out, axis=0)
            pool_ref[pl.ds(pl.multiple_of(gg * 8, 8), 8), :] = blk
            return carry

        lax.fori_loop(0, g_num // 8, seg_group, 0)

    return pl.pallas_call(
        body,
        out_shape=(jax.ShapeDtypeStruct((n, nh), jnp.float32),
                   jax.ShapeDtypeStruct((g_num, 2 * nh), jnp.float32)),
        in_specs=[
            pl.BlockSpec(memory_space=pltpu.VMEM),
            pl.BlockSpec(memory_space=pltpu.VMEM),
            pl.BlockSpec(memory_space=pltpu.VMEM),
            pl.BlockSpec(memory_space=pltpu.VMEM),
            pl.BlockSpec(memory_space=pltpu.VMEM),
            pl.BlockSpec(memory_space=pltpu.VMEM),
            pl.BlockSpec(memory_space=pltpu.VMEM),
            pl.BlockSpec(memory_space=pltpu.VMEM),
            pl.BlockSpec(memory_space=pltpu.SMEM),
        ],
        out_specs=(pl.BlockSpec(memory_space=pltpu.VMEM),
                   pl.BlockSpec(memory_space=pltpu.VMEM)),
    )(xin, agg, w1, b1, w2, b2, gamma, beta, starts)


def _head(pools, l1, l2, l3):
    p = pools[0] + pools[1]
    q = pools[2] + pools[3]
    ng = l3['w'].shape[1]
    ng_pad = ((ng + 127) // 128) * 128
    w3 = jnp.pad(l3['w'], ((0, 0), (0, ng_pad - ng)))
    b3 = jnp.pad(l3['b'], (0, ng_pad - ng)).reshape(1, -1)

    def body(p_ref, q_ref, w1_ref, b1_ref, w2_ref, b2_ref, w3_ref, b3_ref,
             y_ref):
        z = p_ref[...] + q_ref[...]
        z = jnp.maximum(
            jnp.dot(z, w1_ref[...], preferred_element_type=jnp.float32)
            + b1_ref[...], 0.0)
        z = jnp.maximum(
            jnp.dot(z, w2_ref[...], preferred_element_type=jnp.float32)
            + b2_ref[...], 0.0)
        y_ref[...] = (jnp.dot(z, w3_ref[...],
                              preferred_element_type=jnp.float32)
                      + b3_ref[...])

    y = pl.pallas_call(
        body,
        out_shape=jax.ShapeDtypeStruct((pools[0].shape[0], ng_pad),
                                       jnp.float32),
    )(p, q, l1['w'], l1['b'].reshape(1, -1), l2['w'], l2['b'].reshape(1, -1),
      w3, b3)
    return y[:, :ng]


def kernel(x, params, edge_index, batch):
    n, d = x.shape
    e = edge_index.shape[1]
    g_num = 128
    epw = e // NSUB
    nfull = (epw // K) // 2 * 2
    ktail = epw - nfull * K
    npad = ((n + 127) // 128) * 128

    src = edge_index[0]
    dst_w = edge_index[1].reshape(NSUB, epw)
    dst_full = dst_w[:, :nfull * K].reshape(NSUB, nfull, K)
    dst_tail = dst_w[:, nfull * K:]
    zeros_tile = jnp.zeros((npad // 16, d), jnp.float32)

    counts = jnp.sum(
        batch[None, :] == jnp.arange(g_num, dtype=jnp.int32)[:, None],
        axis=1, dtype=jnp.int32)
    starts = jnp.concatenate(
        [jnp.zeros((1,), jnp.int32), jnp.cumsum(counts, dtype=jnp.int32)])

    h = x
    pools = []
    for l in range(4):
        cp = params['conv%d' % l]
        bn = params['bn%d' % l]
        agg = _sc_aggregate(h, src, dst_full, dst_tail, zeros_tile)
        h, pool = _tc_layer(h, agg, cp['w1'], cp['b1'].reshape(1, -1),
                            cp['w2'], cp['b2'].reshape(1, -1),
                            bn['gamma'].reshape(1, -1),
                            bn['beta'].reshape(1, -1), starts)
        pools.append(pool)
    return _head(pools, params['lin1'], params['lin2'], params['lin3'])


# speedup vs baseline: 1.2695x; 1.0140x over previous
import functools

import jax
import jax.numpy as jnp
from jax import lax
from jax.experimental import pallas as pl
from jax.experimental.pallas import tpu as pltpu
from jax.experimental.pallas import tpu_sc as plsc

NSUB = 32
K = 104


def _sc_aggregate(h, src, dst_full, dst_tail, zeros_tile):
    n, d = h.shape
    nfull = dst_full.shape[1]
    ktail = dst_tail.shape[1]
    epw = nfull * K + ktail
    npad = zeros_tile.shape[0] * 16
    rows_per_tile = npad // 16
    nlane = 2
    nstep = nfull // nlane

    mesh = plsc.VectorSubcoreMesh(core_axis_name="c", subcore_axis_name="s")

    @functools.partial(
        pl.kernel,
        out_type=jax.ShapeDtypeStruct((2, npad, d), jnp.float32),
        mesh=mesh,
        scratch_types=[
            pltpu.VMEM_SHARED((npad, d), jnp.float32),
            pltpu.VMEM((epw,), jnp.int32),
            pltpu.VMEM((nfull, K), jnp.int32),
            pltpu.VMEM((ktail,), jnp.int32),
            [pltpu.VMEM((K, d), jnp.float32)] * 2,
            [pltpu.SemaphoreType.DMA] * 2,
            [pltpu.SemaphoreType.DMA] * 2,
        ],
    )
    def agg_kernel(h_hbm, src_hbm, dst_hbm, dstt_hbm, z_hbm, out_hbm,
                   acc, src_v, dst_v, dstt_v, rows, gsems, ssems):
        c = lax.axis_index("c")
        s = lax.axis_index("s")
        wid = s * 2 + c
        r0 = pl.multiple_of(s * rows_per_tile, 8)

        pltpu.sync_copy(z_hbm, acc.at[pl.ds(r0, rows_per_tile)])
        pltpu.sync_copy(src_hbm.at[pl.ds(pl.multiple_of(wid * epw, 8), epw)],
                        src_v)
        pltpu.sync_copy(dst_hbm.at[wid], dst_v)
        pltpu.sync_copy(dstt_hbm.at[wid], dstt_v)
        plsc.subcore_barrier()

        def gath(j, buf, sem):
            off = pl.multiple_of(j * K, 8)
            pltpu.async_copy(h_hbm.at[src_v.at[pl.ds(off, K)]], buf, sem)

        def wait_g(buf, sem):
            pltpu.make_async_copy(h_hbm.at[src_v.at[pl.ds(0, K)]],
                                  buf, sem).wait()

        def wait_s(buf, sem):
            pltpu.make_async_copy(buf, acc.at[dst_v.at[0]], sem).wait()

        for l in range(nlane):
            gath(l, rows[l], gsems[l])

        def step(j3, carry):
            j0 = j3 * nlane
            for l in range(nlane):
                wait_g(rows[l], gsems[l])
                pltpu.sync_copy(rows[l], acc.at[dst_v.at[j0 + l]], add=True)

                @pl.when(j0 + l + nlane < nfull)
                def _(l=l):
                    gath(j0 + l + nlane, rows[l], gsems[l])
            return carry

        lax.fori_loop(0, nstep, step, 0)

        toff = pl.multiple_of(nfull * K, 8)
        pltpu.async_copy(h_hbm.at[src_v.at[pl.ds(toff, ktail)]],
                         rows[0].at[pl.ds(0, ktail)], gsems[0]).wait()
        pltpu.sync_copy(rows[0].at[pl.ds(0, ktail)], acc.at[dstt_v], add=True)

        plsc.subcore_barrier()
        pltpu.sync_copy(acc.at[pl.ds(r0, rows_per_tile)],
                        out_hbm.at[c].at[pl.ds(r0, rows_per_tile)])

    return agg_kernel(h, src, dst_full, dst_tail, zeros_tile)


def _tc_layer(xin, agg, w1, b1, w2, b2, gamma, beta, starts):
    n, d = xin.shape
    nh = w1.shape[1]
    g_num = starts.shape[0] - 1
    neg_inf = float('-inf')

    def body(x_ref, agg_ref, w1_ref, b1_ref, w2_ref, b2_ref, ga_ref, be_ref,
             st_ref, h_ref, pool_ref):
        hin = x_ref[...] + agg_ref[0, :n, :] + agg_ref[1, :n, :]
        t = jnp.maximum(
            jnp.dot(hin, w1_ref[...], preferred_element_type=jnp.float32)
            + b1_ref[...], 0.0)
        u = (jnp.dot(t, w2_ref[...], preferred_element_type=jnp.float32)
             + b2_ref[...])
        mu = jnp.mean(u, axis=0, keepdims=True)
        var = jnp.mean((u - mu) * (u - mu), axis=0, keepdims=True)
        h = jnp.maximum(
            ga_ref[...] * (u - mu) * lax.rsqrt(var + 1e-5) + be_ref[...], 0.0)
        h_ref[...] = h

        def seg_group(gg, carry):
            rows_out = []
            for k8 in range(8):
                gi = gg * 8 + k8
                s0 = st_ref[gi]
                e0 = st_ref[gi + 1]
                c0 = s0 // 8
                c1 = (e0 + 7) // 8

                def chunk(ci, acc):
                    mx, sm = acc
                    rows = h_ref[pl.ds(pl.multiple_of(ci * 8, 8), 8), :]
                    ridx = ci * 8 + lax.broadcasted_iota(jnp.int32, (8, 1), 0)
                    m = (ridx >= s0) & (ridx < e0)
                    mx = jnp.maximum(mx, jnp.where(m, rows, neg_inf))
                    sm = sm + jnp.where(m, rows, 0.0)
                    return (mx, sm)

                mx, sm = lax.fori_loop(
                    c0, c1, chunk,
                    (jnp.full((8, nh), neg_inf, jnp.float32),
                     jnp.zeros((8, nh), jnp.float32)))
                mxr = jnp.max(mx, axis=0, keepdims=True)
                smr = jnp.sum(sm, axis=0, keepdims=True)
                cnt = jnp.maximum((e0 - s0).astype(jnp.float32), 1.0)
                rows_out.append(jnp.concatenate([mxr, smr / cnt], axis=1))
            blk = jnp.concatenate(rows_out, axis=0)
            pool_ref[pl.ds(pl.multiple_of(gg * 8, 8), 8), :] = blk
            return carry

        lax.fori_loop(0, g_num // 8, seg_group, 0)

    return pl.pallas_call(
        body,
        out_shape=(jax.ShapeDtypeStruct((n, nh), jnp.float32),
                   jax.ShapeDtypeStruct((g_num, 2 * nh), jnp.float32)),
        in_specs=[
            pl.BlockSpec(memory_space=pltpu.VMEM),
            pl.BlockSpec(memory_space=pltpu.VMEM),
            pl.BlockSpec(memory_space=pltpu.VMEM),
            pl.BlockSpec(memory_space=pltpu.VMEM),
            pl.BlockSpec(memory_space=pltpu.VMEM),
            pl.BlockSpec(memory_space=pltpu.VMEM),
            pl.BlockSpec(memory_space=pltpu.VMEM),
            pl.BlockSpec(memory_space=pltpu.VMEM),
            pl.BlockSpec(memory_space=pltpu.SMEM),
        ],
        out_specs=(pl.BlockSpec(memory_space=pltpu.VMEM),
                   pl.BlockSpec(memory_space=pltpu.VMEM)),
    )(xin, agg, w1, b1, w2, b2, gamma, beta, starts)


def _head(pools, l1, l2, l3):
    p = pools[0] + pools[1]
    q = pools[2] + pools[3]
    ng = l3['w'].shape[1]
    ng_pad = ((ng + 127) // 128) * 128
    w3 = jnp.pad(l3['w'], ((0, 0), (0, ng_pad - ng)))
    b3 = jnp.pad(l3['b'], (0, ng_pad - ng)).reshape(1, -1)

    def body(p_ref, q_ref, w1_ref, b1_ref, w2_ref, b2_ref, w3_ref, b3_ref,
             y_ref):
        z = p_ref[...] + q_ref[...]
        z = jnp.maximum(
            jnp.dot(z, w1_ref[...], preferred_element_type=jnp.float32)
            + b1_ref[...], 0.0)
        z = jnp.maximum(
            jnp.dot(z, w2_ref[...], preferred_element_type=jnp.float32)
            + b2_ref[...], 0.0)
        y_ref[...] = (jnp.dot(z, w3_ref[...],
                              preferred_element_type=jnp.float32)
                      + b3_ref[...])

    y = pl.pallas_call(
        body,
        out_shape=jax.ShapeDtypeStruct((pools[0].shape[0], ng_pad),
                                       jnp.float32),
    )(p, q, l1['w'], l1['b'].reshape(1, -1), l2['w'], l2['b'].reshape(1, -1),
      w3, b3)
    return y[:, :ng]


def kernel(x, params, edge_index, batch):
    n, d = x.shape
    e = edge_index.shape[1]
    g_num = 128
    epw = e // NSUB
    nfull = (epw // K) // 2 * 2
    ktail = epw - nfull * K
    npad = ((n + 127) // 128) * 128

    src = edge_index[0]
    dst_w = edge_index[1].reshape(NSUB, epw)
    dst_full = dst_w[:, :nfull * K].reshape(NSUB, nfull, K)
    dst_tail = dst_w[:, nfull * K:]
    zeros_tile = jnp.zeros((npad // 16, d), jnp.float32)

    counts = jnp.sum(
        batch[None, :] == jnp.arange(g_num, dtype=jnp.int32)[:, None],
        axis=1, dtype=jnp.int32)
    starts = jnp.concatenate(
        [jnp.zeros((1,), jnp.int32), jnp.cumsum(counts, dtype=jnp.int32)])

    h = x
    pools = []
    for l in range(4):
        cp = params['conv%d' % l]
        bn = params['bn%d' % l]
        agg = _sc_aggregate(h, src, dst_full, dst_tail, zeros_tile)
        h, pool = _tc_layer(h, agg, cp['w1'], cp['b1'].reshape(1, -1),
                            cp['w2'], cp['b2'].reshape(1, -1),
                            bn['gamma'].reshape(1, -1),
                            bn['beta'].reshape(1, -1), starts)
        pools.append(pool)
    return _head(pools, params['lin1'], params['lin2'], params['lin3'])
